# baseline (speedup 1.0000x reference)
import jax
import jax.numpy as jnp
from jax.experimental import pallas as pl
from jax.experimental.pallas import tpu as pltpu

B = 8
K = 128
N = 100000
BN = 16384
GRID = (N + BN - 1) // BN

_TINY = 1.1754943508222875e-38


def _threefry2x32_bits(idx):
    rot0 = (13, 15, 26, 6)
    rot1 = (17, 29, 16, 24)
    ks0 = jnp.uint32(0)
    ks1 = jnp.uint32(1)
    ks2 = ks0 ^ ks1 ^ jnp.uint32(0x1BD11BDA)
    ks = (ks0, ks1, ks2)

    x0 = jnp.zeros_like(idx) + ks0
    x1 = idx + ks1

    def rotl(x, d):
        return (x << jnp.uint32(d)) | (x >> jnp.uint32(32 - d))

    for blk, rots in enumerate((rot0, rot1, rot0, rot1, rot0)):
        for r in rots:
            x0 = x0 + x1
            x1 = x0 ^ rotl(x1, r)
        x0 = x0 + ks[(blk + 1) % 3]
        x1 = x1 + ks[(blk + 2) % 3] + jnp.uint32(blk + 1)
    return x0 ^ x1


def _sample_kernel(state_ref, wt_ref, b_ref, out_ref, m_sc, i_sc):
    j = pl.program_id(0)

    logits = jax.lax.dot_general(
        state_ref[...], wt_ref[...],
        dimension_numbers=(((1,), (1,)), ((), ())),
        preferred_element_type=jnp.float32)
    logits = logits + b_ref[...]

    col = j * BN + jax.lax.broadcasted_iota(jnp.int32, (B, BN), 1)
    row = jax.lax.broadcasted_iota(jnp.int32, (B, BN), 0)
    lin = (row * N + col).astype(jnp.uint32)
    bits = _threefry2x32_bits(lin)
    fb = pltpu.bitcast((bits >> jnp.uint32(9)) | jnp.uint32(0x3F800000),
                       jnp.float32) - jnp.float32(1.0)
    tiny = jnp.float32(_TINY)
    u = jnp.maximum(tiny, fb + tiny)
    score = -jnp.log(-jnp.log(u)) + logits

    score = jnp.where(col < N, score, -jnp.inf)

    m = jnp.max(score, axis=1, keepdims=True)
    cand = jnp.where(score == m, col, jnp.int32(2**31 - 1))
    idx = jnp.min(cand, axis=1, keepdims=True)

    better = (m > m_sc[...]) | (j == 0)
    m_sc[...] = jnp.where(better, m, m_sc[...])
    i_sc[...] = jnp.where(better, idx, i_sc[...])
    out_ref[...] = i_sc[...]


@jax.jit
def kernel(state, W, b):
    wt = W.T
    b2 = b.reshape(1, N)
    out = pl.pallas_call(
        _sample_kernel,
        grid=(GRID,),
        in_specs=[
            pl.BlockSpec((B, K), lambda j: (0, 0)),
            pl.BlockSpec((BN, K), lambda j: (j, 0)),
            pl.BlockSpec((1, BN), lambda j: (0, j)),
        ],
        out_specs=pl.BlockSpec((B, 1), lambda j: (0, 0)),
        out_shape=jax.ShapeDtypeStruct((B, 1), jnp.int32),
        scratch_shapes=[
            pltpu.VMEM((B, 1), jnp.float32),
            pltpu.VMEM((B, 1), jnp.int32),
        ],
        compiler_params=pltpu.CompilerParams(
            dimension_semantics=("arbitrary",),
        ),
    )(state, wt, b2)
    return out.reshape(B)


# speedup vs baseline: 1.0563x; 1.0563x over previous
import jax
import jax.numpy as jnp
from jax.experimental import pallas as pl
from jax.experimental.pallas import tpu as pltpu

B = 8
K = 128
N = 100000
BN = 33408
GRID = (N + BN - 1) // BN

_TINY = 1.1754943508222875e-38


def _threefry2x32_bits(idx):
    rot0 = (13, 15, 26, 6)
    rot1 = (17, 29, 16, 24)
    ks0 = jnp.uint32(0)
    ks1 = jnp.uint32(1)
    ks2 = ks0 ^ ks1 ^ jnp.uint32(0x1BD11BDA)
    ks = (ks0, ks1, ks2)

    x0 = jnp.zeros_like(idx) + ks0
    x1 = idx + ks1

    def rotl(x, d):
        return (x << jnp.uint32(d)) | (x >> jnp.uint32(32 - d))

    for blk, rots in enumerate((rot0, rot1, rot0, rot1, rot0)):
        for r in rots:
            x0 = x0 + x1
            x1 = x0 ^ rotl(x1, r)
        x0 = x0 + ks[(blk + 1) % 3]
        x1 = x1 + ks[(blk + 2) % 3] + jnp.uint32(blk + 1)
    return x0 ^ x1


def _sample_kernel(state_ref, wt_ref, b_ref, out_ref, m_sc, i_sc):
    j = pl.program_id(0)

    logits = jax.lax.dot_general(
        state_ref[...], wt_ref[...],
        dimension_numbers=(((1,), (1,)), ((), ())),
        preferred_element_type=jnp.float32)
    logits = logits + b_ref[...]

    col = j * BN + jax.lax.broadcasted_iota(jnp.int32, (B, BN), 1)
    row = jax.lax.broadcasted_iota(jnp.int32, (B, BN), 0)
    lin = (row * N + col).astype(jnp.uint32)
    bits = _threefry2x32_bits(lin)
    fb = pltpu.bitcast((bits >> jnp.uint32(9)) | jnp.uint32(0x3F800000),
                       jnp.float32) - jnp.float32(1.0)
    tiny = jnp.float32(_TINY)
    u = jnp.maximum(tiny, fb + tiny)
    score = -jnp.log(-jnp.log(u)) + logits

    score = jnp.where(col < N, score, -jnp.inf)

    m = jnp.max(score, axis=1, keepdims=True)
    cand = jnp.where(score == m, col, jnp.int32(2**31 - 1))
    idx = jnp.min(cand, axis=1, keepdims=True)

    better = (m > m_sc[...]) | (j == 0)
    m_sc[...] = jnp.where(better, m, m_sc[...])
    i_sc[...] = jnp.where(better, idx, i_sc[...])
    out_ref[...] = i_sc[...]


@jax.jit
def kernel(state, W, b):
    wt = W.T
    b2 = b.reshape(1, N)
    out = pl.pallas_call(
        _sample_kernel,
        grid=(GRID,),
        in_specs=[
            pl.BlockSpec((B, K), lambda j: (0, 0)),
            pl.BlockSpec((BN, K), lambda j: (j, 0)),
            pl.BlockSpec((1, BN), lambda j: (0, j)),
        ],
        out_specs=pl.BlockSpec((B, 1), lambda j: (0, 0)),
        out_shape=jax.ShapeDtypeStruct((B, 1), jnp.int32),
        scratch_shapes=[
            pltpu.VMEM((B, 1), jnp.float32),
            pltpu.VMEM((B, 1), jnp.int32),
        ],
        compiler_params=pltpu.CompilerParams(
            dimension_semantics=("arbitrary",),
        ),
    )(state, wt, b2)
    return out.reshape(B)


# speedup vs baseline: 1.0624x; 1.0058x over previous
import jax
import jax.numpy as jnp
from jax.experimental import pallas as pl
from jax.experimental.pallas import tpu as pltpu

B = 8
K = 128
N = 100000
NSPLIT = 2
BSUB = 16768
BN = NSPLIT * BSUB
GRID = (N + BN - 1) // BN

_TINY = 1.1754943508222875e-38


def _threefry2x32_bits(idx):
    rot0 = (13, 15, 26, 6)
    rot1 = (17, 29, 16, 24)
    ks0 = jnp.uint32(0)
    ks1 = jnp.uint32(1)
    ks2 = ks0 ^ ks1 ^ jnp.uint32(0x1BD11BDA)
    ks = (ks0, ks1, ks2)

    x0 = jnp.zeros_like(idx) + ks0
    x1 = idx + ks1

    def rotl(x, d):
        return (x << jnp.uint32(d)) | (x >> jnp.uint32(32 - d))

    for blk, rots in enumerate((rot0, rot1, rot0, rot1, rot0)):
        for r in rots:
            x0 = x0 + x1
            x1 = x0 ^ rotl(x1, r)
        x0 = x0 + ks[(blk + 1) % 3]
        x1 = x1 + ks[(blk + 2) % 3] + jnp.uint32(blk + 1)
    return x0 ^ x1


def _sample_kernel(state_ref, *refs):
    w_refs = refs[:NSPLIT]
    b_refs = refs[NSPLIT:2 * NSPLIT]
    out_ref = refs[2 * NSPLIT]
    m_sc, i_sc = refs[2 * NSPLIT + 1], refs[2 * NSPLIT + 2]
    j = pl.program_id(0)

    for i in range(NSPLIT):
        logits = jax.lax.dot_general(
            state_ref[...], w_refs[i][...],
            dimension_numbers=(((1,), (1,)), ((), ())),
            preferred_element_type=jnp.float32)
        logits = logits + b_refs[i][...]

        base = (j * NSPLIT + i) * BSUB
        col = base + jax.lax.broadcasted_iota(jnp.int32, (B, BSUB), 1)
        row = jax.lax.broadcasted_iota(jnp.int32, (B, BSUB), 0)
        lin = (row * N + col).astype(jnp.uint32)
        bits = _threefry2x32_bits(lin)
        fb = pltpu.bitcast((bits >> jnp.uint32(9)) | jnp.uint32(0x3F800000),
                           jnp.float32) - jnp.float32(1.0)
        tiny = jnp.float32(_TINY)
        u = jnp.maximum(tiny, fb + tiny)
        score = -jnp.log(-jnp.log(u)) + logits

        score = jnp.where(col < N, score, -jnp.inf)

        m = jnp.max(score, axis=1, keepdims=True)
        cand = jnp.where(score == m, col, jnp.int32(2**31 - 1))
        idx = jnp.min(cand, axis=1, keepdims=True)

        if i == 0:
            better = (m > m_sc[...]) | (j == 0)
        else:
            better = m > m_sc[...]
        m_sc[...] = jnp.where(better, m, m_sc[...])
        i_sc[...] = jnp.where(better, idx, i_sc[...])

    out_ref[...] = i_sc[...]


@jax.jit
def kernel(state, W, b):
    wt = W.T
    b2 = b.reshape(1, N)
    w_specs = [
        pl.BlockSpec((BSUB, K), lambda j, i=i: (NSPLIT * j + i, 0))
        for i in range(NSPLIT)
    ]
    b_specs = [
        pl.BlockSpec((1, BSUB), lambda j, i=i: (0, NSPLIT * j + i))
        for i in range(NSPLIT)
    ]
    out = pl.pallas_call(
        _sample_kernel,
        grid=(GRID,),
        in_specs=[pl.BlockSpec((B, K), lambda j: (0, 0))]
                 + w_specs + b_specs,
        out_specs=pl.BlockSpec((B, 1), lambda j: (0, 0)),
        out_shape=jax.ShapeDtypeStruct((B, 1), jnp.int32),
        scratch_shapes=[
            pltpu.VMEM((B, 1), jnp.float32),
            pltpu.VMEM((B, 1), jnp.int32),
        ],
        compiler_params=pltpu.CompilerParams(
            dimension_semantics=("arbitrary",),
        ),
    )(state, *([wt] * NSPLIT), *([b2] * NSPLIT))
    return out.reshape(B)
